# Initial kernel scaffold; baseline (speedup 1.0000x reference)
import jax
import jax.numpy as jnp
from jax.experimental import pallas as pl


def kernel(x, edge_index, W1, b1, W2, b2):
    raise NotImplementedError("write your pallas kernel here")


# speedup vs baseline: 12.8825x; 12.8825x over previous
import functools

import jax
import jax.numpy as jnp
from jax import lax
from jax.experimental import pallas as pl
from jax.experimental.pallas import tpu as pltpu
from jax.experimental.pallas import tpu_sc as plsc

NC = 2
NS = 16
NW = NC * NS
K = 128

_vector_mesh = plsc.VectorSubcoreMesh(core_axis_name="c", subcore_axis_name="s")


def _deg_body(blk, rpt, dst_hbm, deg_out, idx_v, ones_v, z_v, deg_sh):
    c = lax.axis_index("c")
    s = lax.axis_index("s")
    slab = c * NS + s
    z16 = jnp.zeros((16,), jnp.float32)
    o16 = jnp.ones((16,), jnp.float32)
    for i in range(K // 16):
        ones_v[pl.ds(i * 16, 16)] = o16
    for i in range(rpt // 16):
        z_v[pl.ds(i * 16, 16)] = z16
    pltpu.sync_copy(z_v, deg_sh.at[pl.ds(s * rpt, rpt)])
    pltpu.sync_copy(dst_hbm.at[slab], idx_v)
    plsc.subcore_barrier()

    @pl.loop(0, blk)
    def _(j):
        pltpu.sync_copy(ones_v, deg_sh.at[idx_v.at[j]], add=True)

    plsc.subcore_barrier()
    pltpu.sync_copy(deg_sh.at[pl.ds(s * rpt, rpt)],
                    deg_out.at[c, pl.ds(s * rpt, rpt)])


def _agg_body(blk, rpt, d, hw_hbm, src_hbm, dst_hbm, out_hbm,
              src_v, dst_v, buf, z_v, acc_sh):
    c = lax.axis_index("c")
    s = lax.axis_index("s")
    slab = c * NS + s
    z16 = jnp.zeros((16,), jnp.float32)
    for r in range(16):
        for i in range(d // 16):
            z_v[r, pl.ds(i * 16, 16)] = z16

    @pl.loop(0, rpt // 16)
    def _(i):
        pltpu.sync_copy(z_v, acc_sh.at[pl.ds(s * rpt + i * 16, 16)])

    pltpu.sync_copy(src_hbm.at[slab], src_v)
    pltpu.sync_copy(dst_hbm.at[slab], dst_v)
    plsc.subcore_barrier()

    @pl.loop(0, blk)
    def _(j):
        pltpu.sync_copy(hw_hbm.at[src_v.at[j]], buf)
        pltpu.sync_copy(buf, acc_sh.at[dst_v.at[j]], add=True)

    plsc.subcore_barrier()
    pltpu.sync_copy(acc_sh.at[pl.ds(s * rpt, rpt)],
                    out_hbm.at[c, pl.ds(s * rpt, rpt)])


def _deg_kernel(n_pad, blk):
    rpt = n_pad // NS
    return pl.kernel(
        functools.partial(_deg_body, blk, rpt),
        out_type=jax.ShapeDtypeStruct((NC, n_pad), jnp.float32),
        mesh=_vector_mesh,
        scratch_types=[
            pltpu.VMEM((blk, K), jnp.int32),
            pltpu.VMEM((K,), jnp.float32),
            pltpu.VMEM((rpt,), jnp.float32),
            pltpu.VMEM_SHARED((n_pad,), jnp.float32),
        ],
    )


def _agg_kernel(n_pad, blk, d):
    rpt = n_pad // NS
    return pl.kernel(
        functools.partial(_agg_body, blk, rpt, d),
        out_type=jax.ShapeDtypeStruct((NC, n_pad, d), jnp.float32),
        mesh=_vector_mesh,
        scratch_types=[
            pltpu.VMEM((blk, K), jnp.int32),
            pltpu.VMEM((blk, K), jnp.int32),
            pltpu.VMEM((K, d), jnp.float32),
            pltpu.VMEM((16, d), jnp.float32),
            pltpu.VMEM_SHARED((n_pad, d), jnp.float32),
        ],
    )


def _mm_body(x_ref, w_ref, o_ref):
    o_ref[...] = jnp.dot(x_ref[...], w_ref[...],
                         preferred_element_type=jnp.float32)


def _scale_body(degt_ref, xw_ref, o_ref):
    dinv = lax.rsqrt(degt_ref[...].sum(axis=1, keepdims=True) + 1.0)
    o_ref[...] = xw_ref[...] * dinv


def _mid_body(degt_ref, p0_ref, p1_ref, hws_ref, b_ref, w_ref, o_ref):
    dinv = lax.rsqrt(degt_ref[...].sum(axis=1, keepdims=True) + 1.0)
    h = (p0_ref[...] + p1_ref[...] + hws_ref[...]) * dinv + b_ref[...]
    h = jnp.maximum(h, 0.0)
    o_ref[...] = jnp.dot(h, w_ref[...],
                         preferred_element_type=jnp.float32) * dinv


def _final_body(degt_ref, q0_ref, q1_ref, hws_ref, b_ref, o_ref):
    dinv = lax.rsqrt(degt_ref[...].sum(axis=1, keepdims=True) + 1.0)
    o_ref[...] = (q0_ref[...] + q1_ref[...] + hws_ref[...]) * dinv + b_ref[...]


def kernel(x, edge_index, W1, b1, W2, b2):
    n, d = x.shape
    e = edge_index.shape[1]

    blk = -(-e // (NW * K))
    e_pad = NW * blk * K
    n_pad = ((n + NS * 16) // (NS * 16)) * (NS * 16)

    src = edge_index[0].astype(jnp.int32)
    dst = edge_index[1].astype(jnp.int32)
    pad = e_pad - e
    src = jnp.concatenate([src, jnp.zeros((pad,), jnp.int32)])
    dst = jnp.concatenate([dst, jnp.full((pad,), n, jnp.int32)])
    src3 = src.reshape(NW, blk, K)
    dst3 = dst.reshape(NW, blk, K)

    f32 = jnp.float32

    deg_parts = _deg_kernel(n_pad, blk)(dst3)
    xw1 = pl.pallas_call(
        _mm_body, out_shape=jax.ShapeDtypeStruct((n, d), f32))(x, W1)
    degt = deg_parts[:, :n].T

    hws1 = pl.pallas_call(
        _scale_body, out_shape=jax.ShapeDtypeStruct((n, d), f32))(degt, xw1)

    agg = _agg_kernel(n_pad, blk, d)
    p = agg(hws1, src3, dst3)
    hws2 = pl.pallas_call(
        _mid_body, out_shape=jax.ShapeDtypeStruct((n, d), f32))(
            degt, p[0, :n], p[1, :n], hws1, b1.reshape(1, d), W2)

    q = agg(hws2, src3, dst3)
    out = pl.pallas_call(
        _final_body, out_shape=jax.ShapeDtypeStruct((n, d), f32))(
            degt, q[0, :n], q[1, :n], hws2, b2.reshape(1, d))
    return out
